# Initial kernel scaffold; baseline (speedup 1.0000x reference)
import jax
import jax.numpy as jnp
from jax.experimental import pallas as pl


def kernel(x, edge_index, edge_weight, p1, Wih1, Whh1, bih1, bhh1, Winit1, p2, Wih2, Whh2, bih2, bhh2, Winit2, lin1_W, lin1_b, lin2_W, lin2_b):
    raise NotImplementedError("write your pallas kernel here")


# speedup vs baseline: 2.7419x; 2.7419x over previous
import functools

import jax
import jax.numpy as jnp
from jax.experimental import pallas as pl

N = 10000
E = 320000
D = 128

_MM_BLOCK = 2000


def _mm_scale_kernel(x_ref, w_ref, dis_ref, y_ref):
    acc = jnp.dot(x_ref[...], w_ref[...], preferred_element_type=jnp.float32)
    y_ref[...] = dis_ref[...] * acc


def _mm_scale(x, w, dis2d):
    return pl.pallas_call(
        _mm_scale_kernel,
        grid=(N // _MM_BLOCK,),
        in_specs=[
            pl.BlockSpec((_MM_BLOCK, D), lambda i: (i, 0)),
            pl.BlockSpec((D, D), lambda i: (0, 0)),
            pl.BlockSpec((_MM_BLOCK, 1), lambda i: (i, 0)),
        ],
        out_specs=pl.BlockSpec((_MM_BLOCK, D), lambda i: (i, 0)),
        out_shape=jax.ShapeDtypeStruct((N, D), jnp.float32),
    )(x, w, dis2d)


def _gru_cell(xi, h, Wih, Whh, bih, bhh):
    gi = xi @ Wih.T + bih
    gh = h @ Whh.T + bhh
    ir, iz, inn = jnp.split(gi, 3, axis=-1)
    hr, hz, hn = jnp.split(gh, 3, axis=-1)
    r = jax.nn.sigmoid(ir + hr)
    z = jax.nn.sigmoid(iz + hz)
    n = jnp.tanh(inn + r * hn)
    return (1.0 - z) * n + z * h


def _layer_weight(xin, p, Wih, Whh, bih, bhh, Winit):
    score = jnp.tanh((xin @ p) / (jnp.linalg.norm(p) + 1e-16))
    vals, idx = jax.lax.top_k(score, D)
    xt = xin[idx] * vals[:, None]
    return _gru_cell(xt, Winit, Wih, Whh, bih, bhh)


def kernel(x, edge_index, edge_weight, p1, Wih1, Whh1, bih1, bhh1, Winit1,
           p2, Wih2, Whh2, bih2, bhh2, Winit2, lin1_W, lin1_b, lin2_W, lin2_b):
    row, col = edge_index[0], edge_index[1]
    keep = (row != col).astype(jnp.float32)
    ev = edge_weight * keep

    deg = 1.0 + jax.ops.segment_sum(ev, col, num_segments=N)
    dis = jax.lax.rsqrt(deg)
    dis2d = dis[:, None]

    def layer(xin, p, Wih, Whh, bih, bhh, Winit):
        W = _layer_weight(xin, p, Wih, Whh, bih, bhh, Winit)
        y = _mm_scale(xin, W, dis2d)
        acc = jax.ops.segment_sum(ev[:, None] * y[row], col, num_segments=N)
        return jax.nn.relu(dis2d * (acc + y))

    h = layer(x, p1, Wih1, Whh1, bih1, bhh1, Winit1)
    h = layer(h, p2, Wih2, Whh2, bih2, bhh2, Winit2)

    w_head = lin1_W.T @ lin2_W.T
    b_head = lin1_b @ lin2_W.T + lin2_b
    return h @ w_head + b_head
